# baseline (speedup 1.0000x reference)
import jax
import jax.numpy as jnp
from jax import lax
from jax.experimental import pallas as pl
from jax.experimental.pallas import tpu as pltpu

_N_TOKENS = 32768
_E = 8
_LANES = 128
_ROWS = _N_TOKENS * _E // _LANES


def _routing_body(x_ref, o_ref):
    x = x_ref[...]
    lane = lax.broadcasted_iota(jnp.int32, (_ROWS, _LANES), 1)
    m = jnp.where((lane & 7) == 0, x, 0.0)
    m = m + pltpu.roll(m, 1, axis=1)
    m = m + pltpu.roll(m, 2, axis=1)
    m = m + pltpu.roll(m, 4, axis=1)
    gt = jnp.where(x > m, 1.0, 0.0)
    r = lax.broadcasted_iota(jnp.int32, (_LANES, _LANES), 0)
    c = lax.broadcasted_iota(jnp.int32, (_LANES, _LANES), 1)
    grp = jnp.where((r >> 3) == (c >> 3), 1.0, 0.0)
    cnt = jnp.dot(gt, grp)
    o_ref[...] = jnp.where(cnt < 1.5, x, 0.0)


@jax.jit
def kernel(x):
    xr = x.reshape(_ROWS, _LANES)
    out = pl.pallas_call(
        _routing_body,
        out_shape=jax.ShapeDtypeStruct((_ROWS, _LANES), jnp.float32),
        input_output_aliases={0: 0},
    )(xr)
    return out.reshape(_N_TOKENS, _E)
